# Initial kernel scaffold; baseline (speedup 1.0000x reference)
import jax
import jax.numpy as jnp
from jax.experimental import pallas as pl


def kernel(t_embs, token_mask, delta_front, delta_side, delta_back):
    raise NotImplementedError("write your pallas kernel here")


# speedup vs baseline: 2.1171x; 2.1171x over previous
import jax
import jax.numpy as jnp
from jax.experimental import pallas as pl
from jax.experimental.pallas import tpu as pltpu

N_TOKENS = 8192
D = 2048
ALPHA = 1.0
BLOCK = 512


def _delta_kernel(t_ref, m_ref, d_ref, o_ref):
    t = t_ref[:]
    m = m_ref[:]
    f = d_ref[0:1, :]
    s = d_ref[1:2, :]
    b = d_ref[2:3, :]

    ff = jnp.sum(f * f)
    ss = jnp.sum(s * s)
    bb = jnp.sum(b * b)
    fs = jnp.sum(f * s)
    fb = jnp.sum(f * b)
    sb = jnp.sum(s * b)

    df = jnp.sum(t * f, axis=1, keepdims=True)
    ds = jnp.sum(t * s, axis=1, keepdims=True)
    db = jnp.sum(t * b, axis=1, keepdims=True)

    af = df / ff
    a_s = (ds - af * fs) / ss
    ab = (db - af * fb - a_s * sb) / bb

    corr = af * f + a_s * s + (ab - ALPHA) * b
    o_ref[:] = t - m * corr


def kernel(t_embs, token_mask, delta_front, delta_side, delta_back):
    n, d = t_embs.shape
    m = token_mask.astype(jnp.float32).reshape(n, 1)
    dmat = jnp.concatenate(
        [delta_front[None, :], delta_side[None, :], delta_back[None, :]], axis=0
    )
    grid = (n // BLOCK,)
    return pl.pallas_call(
        _delta_kernel,
        grid=grid,
        in_specs=[
            pl.BlockSpec((BLOCK, d), lambda i: (i, 0)),
            pl.BlockSpec((BLOCK, 1), lambda i: (i, 0)),
            pl.BlockSpec((3, d), lambda i: (0, 0)),
        ],
        out_specs=pl.BlockSpec((BLOCK, d), lambda i: (i, 0)),
        out_shape=jax.ShapeDtypeStruct((n, d), t_embs.dtype),
        compiler_params=pltpu.CompilerParams(
            dimension_semantics=("arbitrary",),
        ),
    )(t_embs, m, dmat)
